# baseline (speedup 1.0000x reference)
import jax
import jax.numpy as jnp
from jax import lax
from jax.experimental import pallas as pl
from jax.experimental.pallas import tpu as pltpu

_SRC_X = 2048
_SRC_Y = 2048
_ROWS = 4096
_COLS = 4096

_K = 8
_SLAB = _ROWS // _K


def _body(bt_ref, b_any, o_any, chunk_v, slab_sem, chunk_sem):
    slabs = [
        pltpu.make_async_copy(
            b_any.at[pl.ds(k * _SLAB, _SLAB), :],
            o_any.at[pl.ds(k * _SLAB, _SLAB), :],
            slab_sem,
        )
        for k in range(_K)
    ]
    for c in slabs:
        c.start()

    fetch = pltpu.make_async_copy(
        b_any.at[pl.ds(_SRC_X, 8), pl.ds(_SRC_Y, 128)], chunk_v, chunk_sem)
    fetch.start()
    fetch.wait()
    ri = lax.broadcasted_iota(jnp.int32, (8, 128), 0)
    ci = lax.broadcasted_iota(jnp.int32, (8, 128), 1)
    chunk_v[...] = jnp.where((ri == 0) & (ci == 0), bt_ref[0, 0], chunk_v[...])

    for c in slabs:
        c.wait()

    put = pltpu.make_async_copy(
        chunk_v, o_any.at[pl.ds(_SRC_X, 8), pl.ds(_SRC_Y, 128)], chunk_sem)
    put.start()
    put.wait()


@jax.jit
def _scatter_copy(bt, b2d):
    return pl.pallas_call(
        _body,
        in_specs=[
            pl.BlockSpec(memory_space=pltpu.SMEM),
            pl.BlockSpec(memory_space=pl.ANY),
        ],
        out_specs=pl.BlockSpec(memory_space=pl.ANY),
        out_shape=jax.ShapeDtypeStruct((_ROWS, _COLS), jnp.float32),
        scratch_shapes=[
            pltpu.VMEM((8, 128), jnp.float32),
            pltpu.SemaphoreType.DMA,
            pltpu.SemaphoreType.DMA,
        ],
    )(bt, b2d)


def kernel(B, Bt):
    out = _scatter_copy(Bt, B.reshape(_ROWS, _COLS))
    return out.reshape(B.shape)


# speedup vs baseline: 46.5868x; 46.5868x over previous
import jax
import jax.numpy as jnp
from jax import lax
from jax.experimental import pallas as pl
from jax.experimental.pallas import tpu as pltpu

_SRC_X = 2048
_SRC_Y = 2048
_ROWS = 4096
_COLS = 4096

_R = 256
_G = _ROWS // _R
_TBLK = _SRC_X // _R
_LR = _SRC_X % _R
_LR8 = (_LR // 8) * 8


def _body(bt_ref, b_ref, o_ref):
    o_ref[...] = b_ref[...]

    @pl.when(pl.program_id(0) == _TBLK)
    def _():
        sub = b_ref[pl.ds(_LR8, 8), pl.ds(_SRC_Y, 128)]
        ri = lax.broadcasted_iota(jnp.int32, (8, 128), 0)
        ci = lax.broadcasted_iota(jnp.int32, (8, 128), 1)
        hit = (ri == _LR - _LR8) & (ci == 0)
        o_ref[pl.ds(_LR8, 8), pl.ds(_SRC_Y, 128)] = jnp.where(
            hit, bt_ref[0, 0], sub)


@jax.jit
def _scatter_copy(bt, b2d):
    return pl.pallas_call(
        _body,
        grid=(_G,),
        in_specs=[
            pl.BlockSpec(memory_space=pltpu.SMEM),
            pl.BlockSpec((_R, _COLS), lambda i: (i, 0)),
        ],
        out_specs=pl.BlockSpec((_R, _COLS), lambda i: (i, 0)),
        out_shape=jax.ShapeDtypeStruct((_ROWS, _COLS), jnp.float32),
    )(bt, b2d)


def kernel(B, Bt):
    out = _scatter_copy(Bt, B.reshape(_ROWS, _COLS))
    return out.reshape(B.shape)


# speedup vs baseline: 46.6747x; 1.0019x over previous
import jax
import jax.numpy as jnp
from jax import lax
from jax.experimental import pallas as pl
from jax.experimental.pallas import tpu as pltpu

_SRC_X = 2048
_SRC_Y = 2048
_ROWS = 4096
_COLS = 4096


def _body(bt_ref, b_ref, o_ref):
    ri = lax.broadcasted_iota(jnp.int32, (8, 128), 0)
    ci = lax.broadcasted_iota(jnp.int32, (8, 128), 1)
    o_ref[...] = jnp.where((ri == 0) & (ci == 0), bt_ref[0, 0], b_ref[...])


@jax.jit
def _scatter_copy(bt, b2d):
    return pl.pallas_call(
        _body,
        grid=(1,),
        in_specs=[
            pl.BlockSpec(memory_space=pltpu.SMEM),
            pl.BlockSpec((8, 128), lambda i: (_SRC_X // 8, _SRC_Y // 128)),
        ],
        out_specs=pl.BlockSpec((8, 128), lambda i: (_SRC_X // 8, _SRC_Y // 128)),
        out_shape=jax.ShapeDtypeStruct((_ROWS, _COLS), jnp.float32),
        input_output_aliases={1: 0},
    )(bt, b2d)


def kernel(B, Bt):
    out = _scatter_copy(Bt, B.reshape(_ROWS, _COLS))
    return out.reshape(B.shape)


# speedup vs baseline: 48.1800x; 1.0323x over previous
import jax
import jax.numpy as jnp
from jax import lax
from jax.experimental import pallas as pl
from jax.experimental.pallas import tpu as pltpu

_SRC_X = 2048
_SRC_Y = 2048
_ROWS = 4096
_COLS = 4096

_C = 256
_NCH = _ROWS // _C
_D = 4
_ISRC = _SRC_X // _C
_LR = _SRC_X % _C
_LR8 = (_LR // 8) * 8


def _body(bt_ref, b_any, o_any, *rest):
    bufs = rest[:_D]
    in_sems = rest[_D:2 * _D]
    out_sems = rest[2 * _D:]

    def in_copy(i, d):
        return pltpu.make_async_copy(
            b_any.at[pl.ds(i * _C, _C), :], bufs[d], in_sems[d])

    def out_copy(i, d):
        return pltpu.make_async_copy(
            bufs[d], o_any.at[pl.ds(i * _C, _C), :], out_sems[d])

    for i in range(_D):
        in_copy(i, i).start()

    for i in range(_NCH):
        d = i % _D
        in_copy(i, d).wait()
        if i == _ISRC:
            ri = lax.broadcasted_iota(jnp.int32, (8, 128), 0)
            ci = lax.broadcasted_iota(jnp.int32, (8, 128), 1)
            sub = bufs[d][pl.ds(_LR8, 8), pl.ds(_SRC_Y, 128)]
            bufs[d][pl.ds(_LR8, 8), pl.ds(_SRC_Y, 128)] = jnp.where(
                (ri == _LR - _LR8) & (ci == 0), bt_ref[0, 0], sub)
        out_copy(i, d).start()
        nxt = i + _D
        if nxt < _NCH:
            out_copy(i, d).wait()
            in_copy(nxt, d).start()

    for i in range(_NCH - _D, _NCH):
        out_copy(i, i % _D).wait()


@jax.jit
def _scatter_copy(bt, b2d):
    return pl.pallas_call(
        _body,
        in_specs=[
            pl.BlockSpec(memory_space=pltpu.SMEM),
            pl.BlockSpec(memory_space=pl.ANY),
        ],
        out_specs=pl.BlockSpec(memory_space=pl.ANY),
        out_shape=jax.ShapeDtypeStruct((_ROWS, _COLS), jnp.float32),
        scratch_shapes=(
            [pltpu.VMEM((_C, _COLS), jnp.float32) for _ in range(_D)]
            + [pltpu.SemaphoreType.DMA for _ in range(2 * _D)]
        ),
    )(bt, b2d)


def kernel(B, Bt):
    out = _scatter_copy(Bt, B.reshape(_ROWS, _COLS))
    return out.reshape(B.shape)


# speedup vs baseline: 49.3585x; 1.0245x over previous
import jax
import jax.numpy as jnp
from jax import lax
from jax.experimental import pallas as pl
from jax.experimental.pallas import tpu as pltpu

_SRC_X = 2048
_SRC_Y = 2048
_ROWS = 4096
_COLS = 4096

_C = 512
_NCH = _ROWS // _C
_D = 4
_ISRC = _SRC_X // _C
_LR = _SRC_X % _C
_LR8 = (_LR // 8) * 8


def _body(bt_ref, b_any, o_any, *rest):
    bufs = rest[:_D]
    in_sems = rest[_D:2 * _D]
    out_sems = rest[2 * _D:]

    def in_copy(i, d):
        return pltpu.make_async_copy(
            b_any.at[pl.ds(i * _C, _C), :], bufs[d], in_sems[d])

    def out_copy(i, d):
        return pltpu.make_async_copy(
            bufs[d], o_any.at[pl.ds(i * _C, _C), :], out_sems[d])

    for i in range(_D):
        in_copy(i, i).start()

    for i in range(_NCH):
        d = i % _D
        in_copy(i, d).wait()
        if i == _ISRC:
            ri = lax.broadcasted_iota(jnp.int32, (8, 128), 0)
            ci = lax.broadcasted_iota(jnp.int32, (8, 128), 1)
            sub = bufs[d][pl.ds(_LR8, 8), pl.ds(_SRC_Y, 128)]
            bufs[d][pl.ds(_LR8, 8), pl.ds(_SRC_Y, 128)] = jnp.where(
                (ri == _LR - _LR8) & (ci == 0), bt_ref[0, 0], sub)
        out_copy(i, d).start()
        nxt = i + _D
        if nxt < _NCH:
            out_copy(i, d).wait()
            in_copy(nxt, d).start()

    for i in range(_NCH - _D, _NCH):
        out_copy(i, i % _D).wait()


@jax.jit
def _scatter_copy(bt, b2d):
    return pl.pallas_call(
        _body,
        in_specs=[
            pl.BlockSpec(memory_space=pltpu.SMEM),
            pl.BlockSpec(memory_space=pl.ANY),
        ],
        out_specs=pl.BlockSpec(memory_space=pl.ANY),
        out_shape=jax.ShapeDtypeStruct((_ROWS, _COLS), jnp.float32),
        scratch_shapes=(
            [pltpu.VMEM((_C, _COLS), jnp.float32) for _ in range(_D)]
            + [pltpu.SemaphoreType.DMA for _ in range(2 * _D)]
        ),
    )(bt, b2d)


def kernel(B, Bt):
    out = _scatter_copy(Bt, B.reshape(_ROWS, _COLS))
    return out.reshape(B.shape)
